# Initial kernel scaffold; baseline (speedup 1.0000x reference)
import jax
import jax.numpy as jnp
from jax.experimental import pallas as pl


def kernel(input_ch1, input_ch2, emb, W, b):
    raise NotImplementedError("write your pallas kernel here")


# speedup vs baseline: 86.7352x; 86.7352x over previous
import functools

import jax
import jax.numpy as jnp
from jax import lax
from jax.experimental import pallas as pl
from jax.experimental.pallas import tpu as pltpu
from jax.experimental.pallas import tpu_sc as plsc

B = 4096
L = 200
EMB_DIM = 128
VOCAB = 1000

NC = 2
NS = 16
LANES = 16
NW = NC * NS
ROWS_PER_W = B // NW
GROUPS = ROWS_PER_W // LANES


def _proj_body(emb_ref, wcat_ref, out_ref):
    out_ref[...] = jnp.dot(emb_ref[...], wcat_ref[...],
                           preferred_element_type=jnp.float32)


_proj = pl.pallas_call(
    _proj_body,
    out_shape=jax.ShapeDtypeStruct((VOCAB, 4), jnp.float32),
)


_sc_mesh = plsc.VectorSubcoreMesh(core_axis_name="c", subcore_axis_name="s")


@functools.partial(
    pl.kernel,
    out_type=jax.ShapeDtypeStruct((2 * B,), jnp.float32),
    mesh=_sc_mesh,
    compiler_params=pltpu.CompilerParams(needs_layout_passes=False),
    scratch_types=[
        pltpu.VMEM((4 * VOCAB,), jnp.float32),
        pltpu.VMEM((ROWS_PER_W * L,), jnp.int32),
        pltpu.VMEM((ROWS_PER_W * L,), jnp.int32),
        pltpu.VMEM((2 * ROWS_PER_W,), jnp.float32),
    ],
)
def _sc_gather(table_hbm, idx1_hbm, idx2_hbm, out_hbm,
               table_v, idx1_v, idx2_v, out_v):
    wid = lax.axis_index("s") * NC + lax.axis_index("c")
    base = wid * ROWS_PER_W
    pltpu.sync_copy(table_hbm, table_v)
    pltpu.sync_copy(idx1_hbm.at[pl.ds(base * L, ROWS_PER_W * L)], idx1_v)
    pltpu.sync_copy(idx2_hbm.at[pl.ds(base * L, ROWS_PER_W * L)], idx2_v)
    for g in range(GROUPS):
        row_off = (lax.iota(jnp.int32, LANES) + g * LANES) * L
        zero = jnp.zeros((LANES,), jnp.float32)

        def step(l, carry, row_off=row_off):
            a0, a1 = carry
            col = jnp.full((LANES,), l, jnp.int32)
            vA = plsc.load_gather(idx1_v, [row_off + col])
            vB = plsc.load_gather(idx2_v, [row_off + col])
            pA = vA * 4
            pB = vB * 4 + 2
            g0 = plsc.load_gather(table_v, [pA]) + plsc.load_gather(table_v, [pB])
            g1 = plsc.load_gather(table_v, [pA + 1]) + plsc.load_gather(table_v, [pB + 1])
            return a0 + g0, a1 + g1

        a0, a1 = lax.fori_loop(0, L, step, (zero, zero))
        out_v[pl.ds(g * LANES, LANES)] = a0
        out_v[pl.ds(ROWS_PER_W + g * LANES, LANES)] = a1
    pltpu.sync_copy(out_v.at[pl.ds(0, ROWS_PER_W)],
                    out_hbm.at[pl.ds(base, ROWS_PER_W)])
    pltpu.sync_copy(out_v.at[pl.ds(ROWS_PER_W, ROWS_PER_W)],
                    out_hbm.at[pl.ds(B + base, ROWS_PER_W)])


def _fin_body(s_ref, b_ref, out_ref):
    logits = s_ref[...] * (1.0 / L) + b_ref[...]
    act = jnp.where(logits >= 0, logits, 0.01 * logits)
    m = jnp.max(act, axis=0, keepdims=True)
    lse = m + jnp.log(jnp.sum(jnp.exp(act - m), axis=0, keepdims=True))
    out_ref[...] = act - lse


_fin = pl.pallas_call(
    _fin_body,
    out_shape=jax.ShapeDtypeStruct((2, B), jnp.float32),
)


def kernel(input_ch1, input_ch2, emb, W, b):
    wcat = jnp.concatenate([W[:, :EMB_DIM].T, W[:, EMB_DIM:].T], axis=1)
    t = _proj(emb, wcat).reshape(-1)
    s = _sc_gather(t, input_ch1.reshape(-1), input_ch2.reshape(-1))
    out = _fin(s.reshape(2, B), b.reshape(2, 1))
    return out.T
